# Initial kernel scaffold; baseline (speedup 1.0000x reference)
import jax
import jax.numpy as jnp
from jax.experimental import pallas as pl


def kernel(node_embeddings, batch, W, b):
    raise NotImplementedError("write your pallas kernel here")


# speedup vs baseline: 5.2779x; 5.2779x over previous
import functools

import jax
import jax.numpy as jnp
from jax import lax
from jax.experimental import pallas as pl
from jax.experimental.pallas import tpu as pltpu
from jax.experimental.pallas import tpu_sc as plsc

N = 50000
D = 256
G = 64
L = 16
CB = D // L
NW = 32
RPW = 1568
CHUNK = 112
NCHUNK_FULL = RPW // CHUNK
LAST_W = NW - 1
LAST_FULL = 12
TAIL_ROWS = N - (LAST_W * RPW + LAST_FULL * CHUNK)
NEG_INF = float("-inf")


def _sc_partials_kernel(x_hbm, ids_hbm, psum_hbm, pmax_hbm, pcnt_hbm,
                        x_v, ids_v, sum_v, max_v, cnt_v):
    wid = lax.axis_index("s") * 2 + lax.axis_index("c")
    base = wid * RPW

    zeros16 = jnp.zeros((L,), jnp.float32)
    neg16 = jnp.full((L,), NEG_INF, jnp.float32)
    ones16 = jnp.ones((L,), jnp.float32)

    def init_body(s, _):
        for cb in range(CB):
            sum_v[s, pl.ds(cb * L, L)] = zeros16
            max_v[s, pl.ds(cb * L, L)] = neg16
        cnt_v[s, :] = zeros16
        return 0
    lax.fori_loop(0, G, init_body, 0)

    def accum_rows(nrows):
        def group_body(g, _):
            bvec = ids_v[pl.ds(g * L, L)]
            for j in range(L):
                row = g * L + j
                b = bvec[j]
                for cb in range(CB):
                    x = x_v[row, pl.ds(cb * L, L)]
                    plsc.addupdate(sum_v.at[b, pl.ds(cb * L, L)], x)
                    m = max_v[b, pl.ds(cb * L, L)]
                    max_v[b, pl.ds(cb * L, L)] = jnp.maximum(m, x)
                plsc.addupdate(cnt_v.at[b], ones16)
            return 0
        lax.fori_loop(0, nrows // L, group_body, 0)

    def do_chunk(start, nrows):
        pltpu.sync_copy(x_hbm.at[pl.ds(start, nrows)], x_v.at[pl.ds(0, nrows)])
        pltpu.sync_copy(ids_hbm.at[pl.ds(start, nrows)], ids_v.at[pl.ds(0, nrows)])
        accum_rows(nrows)

    nfull = jnp.where(wid == LAST_W, LAST_FULL, NCHUNK_FULL)

    def chunk_body(c, _):
        do_chunk(base + c * CHUNK, CHUNK)
        return 0
    lax.fori_loop(0, nfull, chunk_body, 0)

    @pl.when(wid == LAST_W)
    def _():
        do_chunk(base + LAST_FULL * CHUNK, TAIL_ROWS)

    pltpu.sync_copy(sum_v, psum_hbm.at[wid])
    pltpu.sync_copy(max_v, pmax_hbm.at[wid])
    pltpu.sync_copy(cnt_v, pcnt_hbm.at[wid])


def _combine_kernel(psum_ref, pmax_ref, pcnt_ref, w_ref, b_ref, out_ref):
    sums = jnp.sum(psum_ref[...], axis=0)
    maxs = jnp.max(pmax_ref[...], axis=0)
    cnts = jnp.sum(pcnt_ref[...], axis=0)[:, 0:1]
    mean = sums / jnp.maximum(cnts, 1.0)
    maxs = jnp.where(maxs == NEG_INF, 0.0, maxs)
    combined = jnp.concatenate([mean, maxs], axis=1)
    proj = lax.dot_general(combined, w_ref[...],
                           (((1,), (1,)), ((), ())),
                           preferred_element_type=jnp.float32)
    out_ref[...] = proj + b_ref[...]


def kernel(node_embeddings, batch, W, b):
    batch = batch.astype(jnp.int32)

    sc = pl.kernel(
        _sc_partials_kernel,
        mesh=plsc.VectorSubcoreMesh(core_axis_name="c", subcore_axis_name="s"),
        out_type=[
            jax.ShapeDtypeStruct((NW, G, D), jnp.float32),
            jax.ShapeDtypeStruct((NW, G, D), jnp.float32),
            jax.ShapeDtypeStruct((NW, G, L), jnp.float32),
        ],
        scratch_types=[
            pltpu.VMEM((CHUNK, D), jnp.float32),
            pltpu.VMEM((CHUNK,), jnp.int32),
            pltpu.VMEM((G, D), jnp.float32),
            pltpu.VMEM((G, D), jnp.float32),
            pltpu.VMEM((G, L), jnp.float32),
        ],
    )
    psum, pmax, pcnt = sc(node_embeddings, batch)

    out = pl.pallas_call(
        _combine_kernel,
        out_shape=jax.ShapeDtypeStruct((G, D), jnp.float32),
    )(psum, pmax, pcnt, W, b.reshape(1, D))
    return out


# speedup vs baseline: 11.9544x; 2.2650x over previous
import jax
import jax.numpy as jnp
from jax import lax
from jax.experimental import pallas as pl
from jax.experimental.pallas import tpu as pltpu
from jax.experimental.pallas import tpu_sc as plsc

N = 50000
D = 256
G = 64
L = 16
CB = D // L
NW = 32
RPW = 1568
CHUNK = 112
NCHUNK_FULL = RPW // CHUNK
LAST_W = NW - 1
LAST_FULL = 12
TAIL_ROWS = N - (LAST_W * RPW + LAST_FULL * CHUNK)
TAIL_GROUPS = TAIL_ROWS // L
NEG_INF = float("-inf")


def _tree_reduce(xs, op):
    while len(xs) > 1:
        xs = [op(xs[2 * i], xs[2 * i + 1]) for i in range(len(xs) // 2)] + \
             (xs[-1:] if len(xs) % 2 else [])
    return xs[0]


def _sc_partials_kernel(x_hbm, ids_hbm, psum_hbm, pmax_hbm, pcnt_hbm,
                        x0, x1, i0, i1, sum_v, max_v, cnt_v,
                        sx0, sx1, si0, si1):
    wid = lax.axis_index("s") * 2 + lax.axis_index("c")
    base = wid * RPW

    zeros16 = jnp.zeros((L,), jnp.float32)
    neg16 = jnp.full((L,), NEG_INF, jnp.float32)
    ones16 = jnp.ones((L,), jnp.float32)
    iota16 = lax.iota(jnp.int32, L)

    def init_body(s, _):
        for cb in range(CB):
            sum_v[pl.ds(s * D + cb * L, L)] = zeros16
            max_v[pl.ds(s * D + cb * L, L)] = neg16
        cnt_v[pl.ds(s * L, L)] = zeros16
        return 0
    lax.fori_loop(0, G, init_body, 0)

    xb = (x0, x1)
    ib = (i0, i1)
    sxb = (sx0, sx1)
    sib = (si0, si1)

    def start(c, k):
        st = base + c * CHUNK
        pltpu.async_copy(x_hbm.at[pl.ds(st, CHUNK)], xb[k], sxb[k])
        pltpu.async_copy(ids_hbm.at[pl.ds(st, CHUNK)],
                         ib[k].at[pl.ds(0, CHUNK)], sib[k])

    def wait(k):
        pltpu.make_async_copy(x_hbm.at[pl.ds(0, CHUNK)], xb[k], sxb[k]).wait()
        pltpu.make_async_copy(ids_hbm.at[pl.ds(0, CHUNK)],
                              ib[k].at[pl.ds(0, CHUNK)], sib[k]).wait()

    def process(x_v, ids_v, ngroups):
        def group_body(g, _):
            row0 = g * L
            bvec = ids_v[pl.ds(row0, L)]
            b0 = bvec[0]
            uniform = b0 == bvec[L - 1]

            def uniform_path():
                for cb in range(CB):
                    xs = [x_v[row0 + j, pl.ds(cb * L, L)] for j in range(L)]
                    s = _tree_reduce(list(xs), jnp.add)
                    m = _tree_reduce(list(xs), jnp.maximum)
                    plsc.addupdate(sum_v.at[pl.ds(b0 * D + cb * L, L)], s)
                    cur = max_v[pl.ds(b0 * D + cb * L, L)]
                    max_v[pl.ds(b0 * D + cb * L, L)] = jnp.maximum(cur, m)
                plsc.addupdate(cnt_v.at[pl.ds(b0 * L, L)],
                               jnp.full((L,), float(L), jnp.float32))

            def rowwise_path():
                def row_body(j, _):
                    row = row0 + j
                    b = ids_v[pl.ds(row, L)][0]
                    for cb in range(CB):
                        x = x_v[row, pl.ds(cb * L, L)]
                        plsc.addupdate(sum_v.at[pl.ds(b * D + cb * L, L)], x)
                        cur = max_v[pl.ds(b * D + cb * L, L)]
                        max_v[pl.ds(b * D + cb * L, L)] = jnp.maximum(cur, x)
                    plsc.addupdate(cnt_v.at[pl.ds(b * L, L)], ones16)
                    return 0
                lax.fori_loop(0, L, row_body, 0)

            lax.cond(uniform, uniform_path, rowwise_path)
            return 0
        lax.fori_loop(0, ngroups, group_body, 0)

    npairs = jnp.where(wid == LAST_W, LAST_FULL // 2, NCHUNK_FULL // 2)
    start(0, 0)

    def pair_body(p, _):
        c0 = 2 * p
        start(c0 + 1, 1)
        wait(0)
        process(x0, i0, CHUNK // L)

        @pl.when(p + 1 < npairs)
        def _():
            start(c0 + 2, 0)
        wait(1)
        process(x1, i1, CHUNK // L)
        return 0
    lax.fori_loop(0, npairs, pair_body, 0)

    @pl.when(wid == LAST_W)
    def _():
        st = base + LAST_FULL * CHUNK
        pltpu.sync_copy(x_hbm.at[pl.ds(st, TAIL_ROWS)],
                        x0.at[pl.ds(0, TAIL_ROWS)])
        pltpu.sync_copy(ids_hbm.at[pl.ds(st, TAIL_ROWS)],
                        i0.at[pl.ds(0, TAIL_ROWS)])
        process(x0, i0, TAIL_GROUPS)

    pltpu.sync_copy(sum_v, psum_hbm.at[wid])
    pltpu.sync_copy(max_v, pmax_hbm.at[wid])
    pltpu.sync_copy(cnt_v, pcnt_hbm.at[wid])


def _combine_kernel(psum_ref, pmax_ref, pcnt_ref, w_ref, b_ref, out_ref):
    sums = jnp.sum(psum_ref[...], axis=0)
    maxs = jnp.max(pmax_ref[...], axis=0)
    cnts = jnp.sum(pcnt_ref[...], axis=0)[:, 0:1]
    mean = sums / jnp.maximum(cnts, 1.0)
    maxs = jnp.where(maxs == NEG_INF, 0.0, maxs)
    combined = jnp.concatenate([mean, maxs], axis=1)
    proj = lax.dot_general(combined, w_ref[...],
                           (((1,), (1,)), ((), ())),
                           preferred_element_type=jnp.float32)
    out_ref[...] = proj + b_ref[...]


def kernel(node_embeddings, batch, W, b):
    batch = batch.astype(jnp.int32)

    sc = pl.kernel(
        _sc_partials_kernel,
        mesh=plsc.VectorSubcoreMesh(core_axis_name="c", subcore_axis_name="s"),
        out_type=[
            jax.ShapeDtypeStruct((NW, G * D), jnp.float32),
            jax.ShapeDtypeStruct((NW, G * D), jnp.float32),
            jax.ShapeDtypeStruct((NW, G * L), jnp.float32),
        ],
        scratch_types=[
            pltpu.VMEM((CHUNK, D), jnp.float32),
            pltpu.VMEM((CHUNK, D), jnp.float32),
            pltpu.VMEM((CHUNK + L,), jnp.int32),
            pltpu.VMEM((CHUNK + L,), jnp.int32),
            pltpu.VMEM((G * D,), jnp.float32),
            pltpu.VMEM((G * D,), jnp.float32),
            pltpu.VMEM((G * L,), jnp.float32),
            pltpu.SemaphoreType.DMA,
            pltpu.SemaphoreType.DMA,
            pltpu.SemaphoreType.DMA,
            pltpu.SemaphoreType.DMA,
        ],
    )
    psum, pmax, pcnt = sc(node_embeddings, batch)

    out = pl.pallas_call(
        _combine_kernel,
        out_shape=jax.ShapeDtypeStruct((G, D), jnp.float32),
    )(psum.reshape(NW, G, D), pmax.reshape(NW, G, D),
      pcnt.reshape(NW, G, L), W, b.reshape(1, D))
    return out


# speedup vs baseline: 13.2186x; 1.1058x over previous
import jax
import jax.numpy as jnp
from jax import lax
from jax.experimental import pallas as pl
from jax.experimental.pallas import tpu as pltpu
from jax.experimental.pallas import tpu_sc as plsc

N = 50000
D = 256
G = 64
L = 16
CB = D // L
NW = 32
RPW = 1568
CHUNK = 112
NCHUNK_FULL = RPW // CHUNK
LAST_W = NW - 1
LAST_FULL = 12
TAIL_ROWS = N - (LAST_W * RPW + LAST_FULL * CHUNK)
TAIL_GROUPS = TAIL_ROWS // L
NEG_INF = float("-inf")


def _tree_reduce(xs, op):
    while len(xs) > 1:
        xs = [op(xs[2 * i], xs[2 * i + 1]) for i in range(len(xs) // 2)] + \
             (xs[-1:] if len(xs) % 2 else [])
    return xs[0]


def _sc_partials_kernel(x_hbm, ids_hbm, psum_hbm, pmax_hbm, pcnt_hbm,
                        x0, x1, i0, i1, sum_v, max_v, cnt_v,
                        sx0, sx1, si0, si1):
    wid = lax.axis_index("s") * 2 + lax.axis_index("c")
    base = wid * RPW

    zeros16 = jnp.zeros((L,), jnp.float32)
    neg16 = jnp.full((L,), NEG_INF, jnp.float32)
    ones16 = jnp.ones((L,), jnp.float32)
    iota16 = lax.iota(jnp.int32, L)

    def init_body(s, _):
        for cb in range(CB):
            sum_v[s, pl.ds(cb * L, L)] = zeros16
            max_v[s, pl.ds(cb * L, L)] = neg16
        cnt_v[s, :] = zeros16
        return 0
    lax.fori_loop(0, G, init_body, 0)

    xb = (x0, x1)
    ib = (i0, i1)
    sxb = (sx0, sx1)
    sib = (si0, si1)

    def start(c, k):
        st = base + c * CHUNK
        pltpu.async_copy(x_hbm.at[pl.ds(st, CHUNK)], xb[k], sxb[k])
        pltpu.async_copy(ids_hbm.at[pl.ds(st, CHUNK)],
                         ib[k].at[pl.ds(0, CHUNK)], sib[k])

    def wait(k):
        pltpu.make_async_copy(x_hbm.at[pl.ds(0, CHUNK)], xb[k], sxb[k]).wait()
        pltpu.make_async_copy(ids_hbm.at[pl.ds(0, CHUNK)],
                              ib[k].at[pl.ds(0, CHUNK)], sib[k]).wait()

    def process(x_v, ids_v, ngroups):
        def group_body(g, _):
            row0 = g * L
            bvec = ids_v[pl.ds(row0, L)]
            b0 = bvec[0]
            uniform = b0 == bvec[L - 1]

            def uniform_path():
                for cb in range(CB):
                    xs = [x_v[row0 + j, pl.ds(cb * L, L)] for j in range(L)]
                    s = _tree_reduce(list(xs), jnp.add)
                    m = _tree_reduce(list(xs), jnp.maximum)
                    plsc.addupdate(sum_v.at[b0, pl.ds(cb * L, L)], s)
                    cur = max_v[b0, pl.ds(cb * L, L)]
                    max_v[b0, pl.ds(cb * L, L)] = jnp.maximum(cur, m)
                plsc.addupdate(cnt_v.at[b0],
                               jnp.full((L,), float(L), jnp.float32))

            def rowwise_path():
                def row_body(j, _):
                    row = row0 + j
                    b = ids_v[pl.ds(row, L)][0]
                    for cb in range(CB):
                        x = x_v[row, pl.ds(cb * L, L)]
                        plsc.addupdate(sum_v.at[b, pl.ds(cb * L, L)], x)
                        cur = max_v[b, pl.ds(cb * L, L)]
                        max_v[b, pl.ds(cb * L, L)] = jnp.maximum(cur, x)
                    plsc.addupdate(cnt_v.at[b], ones16)
                    return 0
                lax.fori_loop(0, L, row_body, 0)

            lax.cond(uniform, uniform_path, rowwise_path)
            return 0
        lax.fori_loop(0, ngroups, group_body, 0)

    npairs = jnp.where(wid == LAST_W, LAST_FULL // 2, NCHUNK_FULL // 2)
    start(0, 0)

    def pair_body(p, _):
        c0 = 2 * p
        start(c0 + 1, 1)
        wait(0)
        process(x0, i0, CHUNK // L)

        @pl.when(p + 1 < npairs)
        def _():
            start(c0 + 2, 0)
        wait(1)
        process(x1, i1, CHUNK // L)
        return 0
    lax.fori_loop(0, npairs, pair_body, 0)

    @pl.when(wid == LAST_W)
    def _():
        st = base + LAST_FULL * CHUNK
        pltpu.sync_copy(x_hbm.at[pl.ds(st, TAIL_ROWS)],
                        x0.at[pl.ds(0, TAIL_ROWS)])
        pltpu.sync_copy(ids_hbm.at[pl.ds(st, TAIL_ROWS)],
                        i0.at[pl.ds(0, TAIL_ROWS)])
        process(x0, i0, TAIL_GROUPS)

    pltpu.sync_copy(sum_v, psum_hbm.at[wid])
    pltpu.sync_copy(max_v, pmax_hbm.at[wid])
    pltpu.sync_copy(cnt_v, pcnt_hbm.at[wid])


def _combine_kernel(psum_ref, pmax_ref, pcnt_ref, w_ref, b_ref, out_ref):
    sums = jnp.sum(psum_ref[...], axis=0)
    maxs = jnp.max(pmax_ref[...], axis=0)
    cnts = jnp.sum(pcnt_ref[...], axis=0)[:, 0:1]
    mean = sums / jnp.maximum(cnts, 1.0)
    maxs = jnp.where(maxs == NEG_INF, 0.0, maxs)
    combined = jnp.concatenate([mean, maxs], axis=1)
    proj = lax.dot_general(combined, w_ref[...],
                           (((1,), (1,)), ((), ())),
                           preferred_element_type=jnp.float32)
    out_ref[...] = proj + b_ref[...]


def kernel(node_embeddings, batch, W, b):
    batch = batch.astype(jnp.int32)

    sc = pl.kernel(
        _sc_partials_kernel,
        mesh=plsc.VectorSubcoreMesh(core_axis_name="c", subcore_axis_name="s"),
        out_type=[
            jax.ShapeDtypeStruct((NW, G, D), jnp.float32),
            jax.ShapeDtypeStruct((NW, G, D), jnp.float32),
            jax.ShapeDtypeStruct((NW, G, L), jnp.float32),
        ],
        scratch_types=[
            pltpu.VMEM((CHUNK, D), jnp.float32),
            pltpu.VMEM((CHUNK, D), jnp.float32),
            pltpu.VMEM((CHUNK + L,), jnp.int32),
            pltpu.VMEM((CHUNK + L,), jnp.int32),
            pltpu.VMEM((G, D), jnp.float32),
            pltpu.VMEM((G, D), jnp.float32),
            pltpu.VMEM((G, L), jnp.float32),
            pltpu.SemaphoreType.DMA,
            pltpu.SemaphoreType.DMA,
            pltpu.SemaphoreType.DMA,
            pltpu.SemaphoreType.DMA,
        ],
    )
    psum, pmax, pcnt = sc(node_embeddings, batch)

    out = pl.pallas_call(
        _combine_kernel,
        out_shape=jax.ShapeDtypeStruct((G, D), jnp.float32),
    )(psum, pmax, pcnt, W, b.reshape(1, D))
    return out


# speedup vs baseline: 14.1694x; 1.0719x over previous
import jax
import jax.numpy as jnp
from jax import lax
from jax.experimental import pallas as pl
from jax.experimental.pallas import tpu as pltpu
from jax.experimental.pallas import tpu_sc as plsc

N = 50000
D = 256
G = 64
L = 16
CB = D // L
NW = 32
RPW = 1568
CHUNK = 112
NCHUNK_FULL = RPW // CHUNK
LAST_W = NW - 1
LAST_FULL = 12
TAIL_ROWS = N - (LAST_W * RPW + LAST_FULL * CHUNK)
TAIL_GROUPS = TAIL_ROWS // L
NEG_INF = float("-inf")

BX = 2000
NSTEPS = N // BX


def _tree_reduce(xs, op):
    while len(xs) > 1:
        xs = [op(xs[2 * i], xs[2 * i + 1]) for i in range(len(xs) // 2)] + \
             (xs[-1:] if len(xs) % 2 else [])
    return xs[0]


def _sc_max_kernel(x_hbm, ids_hbm, pmax_hbm,
                   x0, x1, i0, i1, max_v, sx0, sx1, si0, si1):
    wid = lax.axis_index("s") * 2 + lax.axis_index("c")
    base = wid * RPW

    neg16 = jnp.full((L,), NEG_INF, jnp.float32)

    def init_body(s, _):
        for cb in range(CB):
            max_v[s, pl.ds(cb * L, L)] = neg16
        return 0
    lax.fori_loop(0, G, init_body, 0)

    xb = (x0, x1)
    ib = (i0, i1)
    sxb = (sx0, sx1)
    sib = (si0, si1)

    def start(c, k):
        st = base + c * CHUNK
        pltpu.async_copy(x_hbm.at[pl.ds(st, CHUNK)], xb[k], sxb[k])
        pltpu.async_copy(ids_hbm.at[pl.ds(st, CHUNK)],
                         ib[k].at[pl.ds(0, CHUNK)], sib[k])

    def wait(k):
        pltpu.make_async_copy(x_hbm.at[pl.ds(0, CHUNK)], xb[k], sxb[k]).wait()
        pltpu.make_async_copy(ids_hbm.at[pl.ds(0, CHUNK)],
                              ib[k].at[pl.ds(0, CHUNK)], sib[k]).wait()

    def process(x_v, ids_v, ngroups):
        def group_body(g, _):
            row0 = g * L
            bvec = ids_v[pl.ds(row0, L)]
            b0 = bvec[0]
            uniform = b0 == bvec[L - 1]

            def uniform_path():
                for cb in range(CB):
                    xs = [x_v[row0 + j, pl.ds(cb * L, L)] for j in range(L)]
                    m = _tree_reduce(list(xs), jnp.maximum)
                    cur = max_v[b0, pl.ds(cb * L, L)]
                    max_v[b0, pl.ds(cb * L, L)] = jnp.maximum(cur, m)

            def rowwise_path():
                def row_body(j, _):
                    row = row0 + j
                    b = ids_v[pl.ds(row, L)][0]
                    for cb in range(CB):
                        x = x_v[row, pl.ds(cb * L, L)]
                        cur = max_v[b, pl.ds(cb * L, L)]
                        max_v[b, pl.ds(cb * L, L)] = jnp.maximum(cur, x)
                    return 0
                lax.fori_loop(0, L, row_body, 0)

            lax.cond(uniform, uniform_path, rowwise_path)
            return 0
        lax.fori_loop(0, ngroups, group_body, 0)

    npairs = jnp.where(wid == LAST_W, LAST_FULL // 2, NCHUNK_FULL // 2)
    start(0, 0)

    def pair_body(p, _):
        c0 = 2 * p
        start(c0 + 1, 1)
        wait(0)
        process(x0, i0, CHUNK // L)

        @pl.when(p + 1 < npairs)
        def _():
            start(c0 + 2, 0)
        wait(1)
        process(x1, i1, CHUNK // L)
        return 0
    lax.fori_loop(0, npairs, pair_body, 0)

    @pl.when(wid == LAST_W)
    def _():
        st = base + LAST_FULL * CHUNK
        pltpu.sync_copy(x_hbm.at[pl.ds(st, TAIL_ROWS)],
                        x0.at[pl.ds(0, TAIL_ROWS)])
        pltpu.sync_copy(ids_hbm.at[pl.ds(st, TAIL_ROWS)],
                        i0.at[pl.ds(0, TAIL_ROWS)])
        process(x0, i0, TAIL_GROUPS)

    pltpu.sync_copy(max_v, pmax_hbm.at[wid])


def _seg_sum_kernel(ids_ref, x_ref, sum_ref, cnt_ref):
    i = pl.program_id(0)
    ids = ids_ref[0, 0, :]
    seg = lax.broadcasted_iota(jnp.int32, (G, BX), 0)
    oh = (seg == ids[None, :]).astype(jnp.bfloat16)
    x = x_ref[...]
    hi = x.astype(jnp.bfloat16)
    lo = (x - hi.astype(jnp.float32)).astype(jnp.bfloat16)
    dn = (((1,), (0,)), ((), ()))
    part = (lax.dot_general(oh, hi, dn, preferred_element_type=jnp.float32) +
            lax.dot_general(oh, lo, dn, preferred_element_type=jnp.float32))
    cpart = jnp.sum(oh.astype(jnp.float32), axis=1, keepdims=True)

    @pl.when(i == 0)
    def _():
        sum_ref[...] = jnp.zeros_like(sum_ref)
        cnt_ref[...] = jnp.zeros_like(cnt_ref)

    sum_ref[...] += part
    cnt_ref[...] += cpart


def _combine_kernel(sum_ref, cnt_ref, pmax_ref, w_ref, b_ref, out_ref):
    maxs = jnp.max(pmax_ref[...], axis=0)
    mean = sum_ref[...] / jnp.maximum(cnt_ref[...], 1.0)
    maxs = jnp.where(maxs == NEG_INF, 0.0, maxs)
    combined = jnp.concatenate([mean, maxs], axis=1)
    proj = lax.dot_general(combined, w_ref[...],
                           (((1,), (1,)), ((), ())),
                           preferred_element_type=jnp.float32)
    out_ref[...] = proj + b_ref[...]


def kernel(node_embeddings, batch, W, b):
    batch = batch.astype(jnp.int32)

    sc = pl.kernel(
        _sc_max_kernel,
        mesh=plsc.VectorSubcoreMesh(core_axis_name="c", subcore_axis_name="s"),
        out_type=[
            jax.ShapeDtypeStruct((NW, G, D), jnp.float32),
        ],
        scratch_types=[
            pltpu.VMEM((CHUNK, D), jnp.float32),
            pltpu.VMEM((CHUNK, D), jnp.float32),
            pltpu.VMEM((CHUNK + L,), jnp.int32),
            pltpu.VMEM((CHUNK + L,), jnp.int32),
            pltpu.VMEM((G, D), jnp.float32),
            pltpu.SemaphoreType.DMA,
            pltpu.SemaphoreType.DMA,
            pltpu.SemaphoreType.DMA,
            pltpu.SemaphoreType.DMA,
        ],
    )
    (pmax,) = sc(node_embeddings, batch)

    sums, cnts = pl.pallas_call(
        _seg_sum_kernel,
        grid=(NSTEPS,),
        in_specs=[
            pl.BlockSpec((1, 1, BX), lambda i: (i, 0, 0)),
            pl.BlockSpec((BX, D), lambda i: (i, 0)),
        ],
        out_specs=[
            pl.BlockSpec((G, D), lambda i: (0, 0)),
            pl.BlockSpec((G, 1), lambda i: (0, 0)),
        ],
        out_shape=[
            jax.ShapeDtypeStruct((G, D), jnp.float32),
            jax.ShapeDtypeStruct((G, 1), jnp.float32),
        ],
        compiler_params=pltpu.CompilerParams(
            dimension_semantics=("arbitrary",)),
    )(batch.reshape(NSTEPS, 1, BX), node_embeddings)

    out = pl.pallas_call(
        _combine_kernel,
        out_shape=jax.ShapeDtypeStruct((G, D), jnp.float32),
    )(sums, cnts, pmax, W, b.reshape(1, D))
    return out
